# Initial kernel scaffold; baseline (speedup 1.0000x reference)
import jax
import jax.numpy as jnp
from jax.experimental import pallas as pl

DIM = 4096
N_EXP = 64
TOPK = 8
N_GROUPS = 8
GROUP_SIZE = N_EXP // N_GROUPS
TOPK_GROUPS = 4
ROUTE_SCALE = 2.5
N_TOK = 32768

BLOCK_T = 256


def _gate_body(x_ref, w_ref, wout_ref, iout_ref):
    T = x_ref.shape[0]
    xb = x_ref[...]
    wb = w_ref[...]
    logits = jax.lax.dot_general(
        xb, wb, (((1,), (1,)), ((), ())), preferred_element_type=jnp.float32
    )
    scores = jax.nn.sigmoid(logits)

    gs = jnp.concatenate(
        [
            jnp.max(scores[:, g * GROUP_SIZE:(g + 1) * GROUP_SIZE], axis=1, keepdims=True)
            for g in range(N_GROUPS)
        ],
        axis=1,
    )

    lane_g = jax.lax.broadcasted_iota(jnp.int32, (T, N_GROUPS), 1)
    gwork = gs
    sel = None
    for _ in range(TOPK_GROUPS):
        m = jnp.max(gwork, axis=1, keepdims=True)
        gidx = jnp.min(jnp.where(gwork == m, lane_g, N_GROUPS), axis=1, keepdims=True)
        pick = lane_g == gidx
        sel = pick if sel is None else (sel | pick)
        gwork = jnp.where(pick, -1.0, gwork)

    mask64 = jnp.concatenate(
        [jnp.broadcast_to(sel[:, g:g + 1], (T, GROUP_SIZE)) for g in range(N_GROUPS)],
        axis=1,
    )
    masked = jnp.where(mask64, scores, 0.0)

    lane_e = jax.lax.broadcasted_iota(jnp.int32, (T, N_EXP), 1)
    mwork = masked
    idx_cols = []
    w_cols = []
    for _ in range(TOPK):
        m = jnp.max(mwork, axis=1, keepdims=True)
        eidx = jnp.min(jnp.where(mwork == m, lane_e, N_EXP), axis=1, keepdims=True)
        pick = lane_e == eidx
        idx_cols.append(eidx)
        w_cols.append(jnp.max(jnp.where(pick, scores, -1.0), axis=1, keepdims=True))
        mwork = jnp.where(pick, -1.0, mwork)

    indices = jnp.concatenate(idx_cols, axis=1)
    w = jnp.concatenate(w_cols, axis=1)
    w = w / jnp.sum(w, axis=1, keepdims=True)
    w = w * ROUTE_SCALE

    wout_ref[...] = w
    iout_ref[...] = indices


def kernel(x, W):
    n_tok = x.shape[0]
    grid = (n_tok // BLOCK_T,)
    wout, iout = pl.pallas_call(
        _gate_body,
        grid=grid,
        in_specs=[
            pl.BlockSpec((BLOCK_T, DIM), lambda i: (i, 0)),
            pl.BlockSpec((N_EXP, DIM), lambda i: (0, 0)),
        ],
        out_specs=[
            pl.BlockSpec((BLOCK_T, TOPK), lambda i: (i, 0)),
            pl.BlockSpec((BLOCK_T, TOPK), lambda i: (i, 0)),
        ],
        out_shape=[
            jax.ShapeDtypeStruct((n_tok, TOPK), jnp.float32),
            jax.ShapeDtypeStruct((n_tok, TOPK), jnp.int32),
        ],
    )(x, W)
    return wout, iout


# speedup vs baseline: 1.5839x; 1.5839x over previous
import jax
import jax.numpy as jnp
from jax.experimental import pallas as pl

DIM = 4096
N_EXP = 64
TOPK = 8
N_GROUPS = 8
GROUP_SIZE = N_EXP // N_GROUPS
TOPK_GROUPS = 4
ROUTE_SCALE = 2.5
N_TOK = 32768

BLOCK_T = 256


def _gate_body(x_ref, w_ref, wout_ref, iout_ref):
    T = x_ref.shape[0]
    xb = x_ref[...]
    wb = w_ref[...]
    logits = jax.lax.dot_general(
        xb, wb, (((1,), (1,)), ((), ())), preferred_element_type=jnp.float32
    )
    scores = jax.nn.sigmoid(logits)

    gs = jnp.concatenate(
        [
            jnp.max(scores[:, g * GROUP_SIZE:(g + 1) * GROUP_SIZE], axis=1, keepdims=True)
            for g in range(N_GROUPS)
        ],
        axis=1,
    )

    lane_g = jax.lax.broadcasted_iota(jnp.int32, (T, N_GROUPS), 1)
    gwork = gs
    sel = jnp.zeros((T, N_GROUPS), jnp.float32)
    for _ in range(TOPK_GROUPS):
        m = jnp.max(gwork, axis=1, keepdims=True)
        gidx = jnp.min(jnp.where(gwork == m, lane_g, N_GROUPS), axis=1, keepdims=True)
        pick = lane_g == gidx
        sel = jnp.where(pick, 1.0, sel)
        gwork = jnp.where(pick, -1.0, gwork)

    mask64 = jnp.concatenate(
        [jnp.broadcast_to(sel[:, g:g + 1], (T, GROUP_SIZE)) for g in range(N_GROUPS)],
        axis=1,
    )
    masked = scores * mask64

    lane_e = jax.lax.broadcasted_iota(jnp.int32, (T, N_EXP), 1)
    mwork = masked
    idx_cols = []
    w_cols = []
    for _ in range(TOPK):
        m = jnp.max(mwork, axis=1, keepdims=True)
        eidx = jnp.min(jnp.where(mwork == m, lane_e, N_EXP), axis=1, keepdims=True)
        pick = lane_e == eidx
        idx_cols.append(eidx)
        w_cols.append(jnp.max(jnp.where(pick, scores, -1.0), axis=1, keepdims=True))
        mwork = jnp.where(pick, -1.0, mwork)

    indices = jnp.concatenate(idx_cols, axis=1)
    w = jnp.concatenate(w_cols, axis=1)
    w = w / jnp.sum(w, axis=1, keepdims=True)
    w = w * ROUTE_SCALE

    wout_ref[...] = w
    iout_ref[...] = indices


def kernel(x, W):
    n_tok = x.shape[0]
    grid = (n_tok // BLOCK_T,)
    wout, iout = pl.pallas_call(
        _gate_body,
        grid=grid,
        in_specs=[
            pl.BlockSpec((BLOCK_T, DIM), lambda i: (i, 0)),
            pl.BlockSpec((N_EXP, DIM), lambda i: (0, 0)),
        ],
        out_specs=[
            pl.BlockSpec((BLOCK_T, TOPK), lambda i: (i, 0)),
            pl.BlockSpec((BLOCK_T, TOPK), lambda i: (i, 0)),
        ],
        out_shape=[
            jax.ShapeDtypeStruct((n_tok, TOPK), jnp.float32),
            jax.ShapeDtypeStruct((n_tok, TOPK), jnp.int32),
        ],
    )(x, W)
    return wout, iout


# speedup vs baseline: 4.0525x; 2.5586x over previous
import jax
import jax.numpy as jnp
from jax.experimental import pallas as pl

DIM = 4096
N_EXP = 64
TOPK = 8
N_GROUPS = 8
GROUP_SIZE = N_EXP // N_GROUPS
TOPK_GROUPS = 4
ROUTE_SCALE = 2.5
N_TOK = 32768

BLOCK_T = 256


def _gate_body(x_ref, w_ref, wout_ref, iout_ref):
    T = x_ref.shape[0]
    xb = x_ref[...]
    wb = w_ref[...]
    logits_t = jax.lax.dot_general(
        wb, xb, (((1,), (1,)), ((), ())), preferred_element_type=jnp.float32
    )
    scores = jax.nn.sigmoid(logits_t)

    gs = jnp.concatenate(
        [
            jnp.max(scores[g * GROUP_SIZE:(g + 1) * GROUP_SIZE, :], axis=0, keepdims=True)
            for g in range(N_GROUPS)
        ],
        axis=0,
    )

    sub_g = jax.lax.broadcasted_iota(jnp.int32, (N_GROUPS, T), 0)
    gwork = gs
    sel = jnp.zeros((N_GROUPS, T), jnp.float32)
    for _ in range(TOPK_GROUPS):
        m = jnp.max(gwork, axis=0, keepdims=True)
        gidx = jnp.min(jnp.where(gwork == m, sub_g, N_GROUPS), axis=0, keepdims=True)
        pick = sub_g == gidx
        sel = jnp.where(pick, 1.0, sel)
        gwork = jnp.where(pick, -1.0, gwork)

    mask64 = jnp.concatenate(
        [jnp.broadcast_to(sel[g:g + 1, :], (GROUP_SIZE, T)) for g in range(N_GROUPS)],
        axis=0,
    )
    masked = scores * mask64

    sub_e = jax.lax.broadcasted_iota(jnp.int32, (N_EXP, T), 0)
    mwork = masked
    idx_rows = []
    w_rows = []
    for _ in range(TOPK):
        m = jnp.max(mwork, axis=0, keepdims=True)
        eidx = jnp.min(jnp.where(mwork == m, sub_e, N_EXP), axis=0, keepdims=True)
        pick = sub_e == eidx
        idx_rows.append(eidx)
        w_rows.append(m)
        mwork = jnp.where(pick, -1.0, mwork)

    idx_t = jnp.concatenate(idx_rows, axis=0)
    w_t = jnp.concatenate(w_rows, axis=0)
    w_t = w_t / jnp.sum(w_t, axis=0, keepdims=True)
    w_t = w_t * ROUTE_SCALE

    wout_ref[...] = w_t
    iout_ref[...] = idx_t


def kernel(x, W):
    n_tok = x.shape[0]
    grid = (n_tok // BLOCK_T,)
    wout_t, iout_t = pl.pallas_call(
        _gate_body,
        grid=grid,
        in_specs=[
            pl.BlockSpec((BLOCK_T, DIM), lambda i: (i, 0)),
            pl.BlockSpec((N_EXP, DIM), lambda i: (0, 0)),
        ],
        out_specs=[
            pl.BlockSpec((TOPK, BLOCK_T), lambda i: (0, i)),
            pl.BlockSpec((TOPK, BLOCK_T), lambda i: (0, i)),
        ],
        out_shape=[
            jax.ShapeDtypeStruct((TOPK, n_tok), jnp.float32),
            jax.ShapeDtypeStruct((TOPK, n_tok), jnp.int32),
        ],
    )(x, W)
    return wout_t.T, iout_t.T


# speedup vs baseline: 5.0017x; 1.2342x over previous
import jax
import jax.numpy as jnp
from jax.experimental import pallas as pl

DIM = 4096
N_EXP = 64
TOPK = 8
N_GROUPS = 8
GROUP_SIZE = N_EXP // N_GROUPS
TOPK_GROUPS = 4
ROUTE_SCALE = 2.5
N_TOK = 32768

BLOCK_T = 512


def _gate_body(x_ref, w_ref, wout_ref, iout_ref):
    T = x_ref.shape[0]
    xb = x_ref[...]
    wb = w_ref[...]
    logits_t = jax.lax.dot_general(
        wb, xb, (((1,), (1,)), ((), ())), preferred_element_type=jnp.float32
    )
    scores = jax.nn.sigmoid(logits_t)

    gs = jnp.concatenate(
        [
            jnp.max(scores[g * GROUP_SIZE:(g + 1) * GROUP_SIZE, :], axis=0, keepdims=True)
            for g in range(N_GROUPS)
        ],
        axis=0,
    )

    sub_g = jax.lax.broadcasted_iota(jnp.int32, (N_GROUPS, T), 0)
    gwork = gs
    sel = jnp.zeros((N_GROUPS, T), jnp.float32)
    for _ in range(TOPK_GROUPS):
        m = jnp.max(gwork, axis=0, keepdims=True)
        gidx = jnp.min(jnp.where(gwork == m, sub_g, N_GROUPS), axis=0, keepdims=True)
        pick = sub_g == gidx
        sel = jnp.where(pick, 1.0, sel)
        gwork = jnp.where(pick, -1.0, gwork)

    mask64 = jnp.concatenate(
        [jnp.broadcast_to(sel[g:g + 1, :], (GROUP_SIZE, T)) for g in range(N_GROUPS)],
        axis=0,
    )
    masked = scores * mask64

    sub_e = jax.lax.broadcasted_iota(jnp.int32, (N_EXP, T), 0)
    mwork = masked
    idx_rows = []
    w_rows = []
    for _ in range(TOPK):
        m = jnp.max(mwork, axis=0, keepdims=True)
        eidx = jnp.min(jnp.where(mwork == m, sub_e, N_EXP), axis=0, keepdims=True)
        pick = sub_e == eidx
        idx_rows.append(eidx)
        w_rows.append(m)
        mwork = jnp.where(pick, -1.0, mwork)

    idx_t = jnp.concatenate(idx_rows, axis=0)
    w_t = jnp.concatenate(w_rows, axis=0)
    w_t = w_t / jnp.sum(w_t, axis=0, keepdims=True)
    w_t = w_t * ROUTE_SCALE

    wout_ref[...] = w_t
    iout_ref[...] = idx_t


def kernel(x, W):
    n_tok = x.shape[0]
    grid = (n_tok // BLOCK_T,)
    wout_t, iout_t = pl.pallas_call(
        _gate_body,
        grid=grid,
        in_specs=[
            pl.BlockSpec((BLOCK_T, DIM), lambda i: (i, 0)),
            pl.BlockSpec((N_EXP, DIM), lambda i: (0, 0)),
        ],
        out_specs=[
            pl.BlockSpec((TOPK, BLOCK_T), lambda i: (0, i)),
            pl.BlockSpec((TOPK, BLOCK_T), lambda i: (0, i)),
        ],
        out_shape=[
            jax.ShapeDtypeStruct((TOPK, n_tok), jnp.float32),
            jax.ShapeDtypeStruct((TOPK, n_tok), jnp.int32),
        ],
    )(x, W)
    return wout_t.T, iout_t.T


# speedup vs baseline: 5.5287x; 1.1054x over previous
import jax
import jax.numpy as jnp
from jax.experimental import pallas as pl

DIM = 4096
N_EXP = 64
TOPK = 8
N_GROUPS = 8
GROUP_SIZE = N_EXP // N_GROUPS
TOPK_GROUPS = 4
ROUTE_SCALE = 2.5
N_TOK = 32768

BLOCK_T = 1024


def _gate_body(x_ref, w_ref, wout_ref, iout_ref):
    T = x_ref.shape[0]
    xb = x_ref[...]
    wb = w_ref[...]
    logits_t = jax.lax.dot_general(
        wb, xb, (((1,), (1,)), ((), ())), preferred_element_type=jnp.float32
    )
    scores = jax.nn.sigmoid(logits_t)

    gs = jnp.concatenate(
        [
            jnp.max(scores[g * GROUP_SIZE:(g + 1) * GROUP_SIZE, :], axis=0, keepdims=True)
            for g in range(N_GROUPS)
        ],
        axis=0,
    )

    sub_g = jax.lax.broadcasted_iota(jnp.int32, (N_GROUPS, T), 0)
    gwork = gs
    sel = jnp.zeros((N_GROUPS, T), jnp.float32)
    for _ in range(TOPK_GROUPS):
        m = jnp.max(gwork, axis=0, keepdims=True)
        gidx = jnp.min(jnp.where(gwork == m, sub_g, N_GROUPS), axis=0, keepdims=True)
        pick = sub_g == gidx
        sel = jnp.where(pick, 1.0, sel)
        gwork = jnp.where(pick, -1.0, gwork)

    mask64 = jnp.concatenate(
        [jnp.broadcast_to(sel[g:g + 1, :], (GROUP_SIZE, T)) for g in range(N_GROUPS)],
        axis=0,
    )
    masked = scores * mask64

    sub_e = jax.lax.broadcasted_iota(jnp.int32, (N_EXP, T), 0)
    mwork = masked
    idx_rows = []
    w_rows = []
    for _ in range(TOPK):
        m = jnp.max(mwork, axis=0, keepdims=True)
        eidx = jnp.min(jnp.where(mwork == m, sub_e, N_EXP), axis=0, keepdims=True)
        pick = sub_e == eidx
        idx_rows.append(eidx)
        w_rows.append(m)
        mwork = jnp.where(pick, -1.0, mwork)

    idx_t = jnp.concatenate(idx_rows, axis=0)
    w_t = jnp.concatenate(w_rows, axis=0)
    w_t = w_t / jnp.sum(w_t, axis=0, keepdims=True)
    w_t = w_t * ROUTE_SCALE

    wout_ref[...] = w_t
    iout_ref[...] = idx_t


def kernel(x, W):
    n_tok = x.shape[0]
    grid = (n_tok // BLOCK_T,)
    wout_t, iout_t = pl.pallas_call(
        _gate_body,
        grid=grid,
        in_specs=[
            pl.BlockSpec((BLOCK_T, DIM), lambda i: (i, 0)),
            pl.BlockSpec((N_EXP, DIM), lambda i: (0, 0)),
        ],
        out_specs=[
            pl.BlockSpec((TOPK, BLOCK_T), lambda i: (0, i)),
            pl.BlockSpec((TOPK, BLOCK_T), lambda i: (0, i)),
        ],
        out_shape=[
            jax.ShapeDtypeStruct((TOPK, n_tok), jnp.float32),
            jax.ShapeDtypeStruct((TOPK, n_tok), jnp.int32),
        ],
    )(x, W)
    return wout_t.T, iout_t.T
